# initial kernel scaffold (unmeasured)
import jax
import jax.numpy as jnp
from jax import lax
from jax.experimental import pallas as pl
from jax.experimental.pallas import tpu as pltpu

M = 8192
D = 2048
HALF = M // 2
R = 512
K = HALF // R


def kernel(partial, resid, gamma):
    my_y = lax.axis_index("y")
    mypart = lax.dynamic_slice(partial[0], (my_y * HALF, 0), (HALF, D))
    myresid = lax.dynamic_slice(resid, (my_y * HALF, 0), (HALF, D))
    gamma2 = gamma.reshape(1, D)

    def body(part_ref, resid_ref, gamma_ref, out_ref,
             recv_x, out_vmem, sx, rx, sy, ry, local_sem):
        k = pl.program_id(0)
        x = lax.axis_index("x")
        y = lax.axis_index("y")
        slot = lax.rem(k, 2)

        @pl.when(k == 0)
        def _():
            bsem = pltpu.get_barrier_semaphore()
            pl.semaphore_signal(bsem, inc=1, device_id=(1 - x, y),
                                device_id_type=pl.DeviceIdType.MESH)
            pl.semaphore_signal(bsem, inc=1, device_id=(x, 1 - y),
                                device_id_type=pl.DeviceIdType.MESH)
            pl.semaphore_wait(bsem, 2)

        rdma_x = pltpu.make_async_remote_copy(
            src_ref=part_ref,
            dst_ref=recv_x.at[slot],
            send_sem=sx.at[k],
            recv_sem=rx.at[k],
            device_id=(1 - x, y),
            device_id_type=pl.DeviceIdType.MESH,
        )
        rdma_x.start()
        rdma_x.wait()

        yc = part_ref[...] + recv_x[slot] + resid_ref[...]
        rms = jnp.sqrt(jnp.mean(yc * yc, axis=-1, keepdims=True) + 1e-6)
        out_vmem[...] = yc / rms * gamma_ref[...]

        my_row = y * HALF + k * R
        their_row = (1 - y) * HALF + k * R

        cp = pltpu.make_async_copy(out_vmem, out_ref.at[pl.ds(my_row, R)],
                                   local_sem)
        cp.start()

        rdma_y = pltpu.make_async_remote_copy(
            src_ref=out_vmem,
            dst_ref=out_ref.at[pl.ds(my_row, R)],
            send_sem=sy.at[k],
            recv_sem=ry.at[k],
            device_id=(x, 1 - y),
            device_id_type=pl.DeviceIdType.MESH,
        )
        rdma_y.start()

        recv_y = pltpu.make_async_remote_copy(
            src_ref=out_vmem,
            dst_ref=out_ref.at[pl.ds(their_row, R)],
            send_sem=sy.at[k],
            recv_sem=ry.at[k],
            device_id=(x, 1 - y),
            device_id_type=pl.DeviceIdType.MESH,
        )
        recv_y.wait_recv()
        rdma_y.wait_send()
        cp.wait()

    return pl.pallas_call(
        body,
        grid=(K,),
        in_specs=[
            pl.BlockSpec((R, D), lambda k: (k, 0)),
            pl.BlockSpec((R, D), lambda k: (k, 0)),
            pl.BlockSpec((1, D), lambda k: (0, 0)),
        ],
        out_specs=pl.BlockSpec(memory_space=pltpu.ANY),
        out_shape=jax.ShapeDtypeStruct((M, D), jnp.float32),
        scratch_shapes=[
            pltpu.VMEM((2, R, D), jnp.float32),
            pltpu.VMEM((R, D), jnp.float32),
            pltpu.SemaphoreType.DMA((K,)),
            pltpu.SemaphoreType.DMA((K,)),
            pltpu.SemaphoreType.DMA((K,)),
            pltpu.SemaphoreType.DMA((K,)),
            pltpu.SemaphoreType.DMA,
        ],
        compiler_params=pltpu.CompilerParams(
            collective_id=0,
            dimension_semantics=("arbitrary",),
        ),
    )(mypart, myresid, gamma2)


# baseline (device time: 853325 ns/iter reference)
import jax
import jax.numpy as jnp
from jax import lax
from jax.experimental import pallas as pl
from jax.experimental.pallas import tpu as pltpu

M = 8192
D = 2048
HALF = M // 2
R = 512
K = HALF // R


def kernel(partial, resid, gamma):
    my_y = lax.axis_index("y")
    mypart = lax.dynamic_slice(partial[0], (my_y * HALF, 0), (HALF, D))
    myresid = lax.dynamic_slice(resid, (my_y * HALF, 0), (HALF, D))
    gamma2 = gamma.reshape(1, D)

    def body(part_ref, resid_ref, gamma_ref, out_ref,
             recv_x, out_vmem, sx, rx, sy, ry, local_sem):
        k = pl.program_id(0)
        x = lax.axis_index("x")
        y = lax.axis_index("y")
        slot = lax.rem(k, 2)

        @pl.when(k == 0)
        def _():
            bsem = pltpu.get_barrier_semaphore()
            pl.semaphore_signal(bsem, inc=1, device_id=(1 - x, y),
                                device_id_type=pl.DeviceIdType.MESH)
            pl.semaphore_signal(bsem, inc=1, device_id=(x, 1 - y),
                                device_id_type=pl.DeviceIdType.MESH)
            pl.semaphore_wait(bsem, 2)

        rdma_x = pltpu.make_async_remote_copy(
            src_ref=part_ref,
            dst_ref=recv_x.at[slot],
            send_sem=sx.at[k],
            recv_sem=rx.at[k],
            device_id=(1 - x, y),
            device_id_type=pl.DeviceIdType.MESH,
        )
        rdma_x.start()
        rdma_x.wait()

        yc = part_ref[...] + recv_x[slot] + resid_ref[...]
        rms = jnp.sqrt(jnp.mean(yc * yc, axis=-1, keepdims=True) + 1e-6)
        out_vmem[...] = yc / rms * gamma_ref[...]

        my_row = y * HALF + k * R
        their_row = (1 - y) * HALF + k * R

        cp = pltpu.make_async_copy(out_vmem, out_ref.at[pl.ds(my_row, R)],
                                   local_sem)
        cp.start()

        rdma_y = pltpu.make_async_remote_copy(
            src_ref=out_vmem,
            dst_ref=out_ref.at[pl.ds(my_row, R)],
            send_sem=sy.at[k],
            recv_sem=ry.at[k],
            device_id=(x, 1 - y),
            device_id_type=pl.DeviceIdType.MESH,
        )
        rdma_y.start()

        recv_y = pltpu.make_async_remote_copy(
            src_ref=out_vmem,
            dst_ref=out_ref.at[pl.ds(their_row, R)],
            send_sem=sy.at[k],
            recv_sem=ry.at[k],
            device_id=(x, 1 - y),
            device_id_type=pl.DeviceIdType.MESH,
        )
        recv_y.wait_recv()
        rdma_y.wait_send()
        cp.wait()

    return pl.pallas_call(
        body,
        grid=(K,),
        in_specs=[
            pl.BlockSpec((R, D), lambda k: (k, 0)),
            pl.BlockSpec((R, D), lambda k: (k, 0)),
            pl.BlockSpec((1, D), lambda k: (0, 0)),
        ],
        out_specs=pl.BlockSpec(memory_space=pl.ANY),
        out_shape=jax.ShapeDtypeStruct((M, D), jnp.float32),
        scratch_shapes=[
            pltpu.VMEM((2, R, D), jnp.float32),
            pltpu.VMEM((R, D), jnp.float32),
            pltpu.SemaphoreType.DMA((K,)),
            pltpu.SemaphoreType.DMA((K,)),
            pltpu.SemaphoreType.DMA((K,)),
            pltpu.SemaphoreType.DMA((K,)),
            pltpu.SemaphoreType.DMA,
        ],
        compiler_params=pltpu.CompilerParams(
            collective_id=0,
            dimension_semantics=("arbitrary",),
            vmem_limit_bytes=64 * 1024 * 1024,
        ),
    )(mypart, myresid, gamma2)


# device time: 506659 ns/iter; 1.6842x vs baseline; 1.6842x over previous
import jax
import jax.numpy as jnp
from jax import lax
from jax.experimental import pallas as pl
from jax.experimental.pallas import tpu as pltpu

M = 8192
D = 2048
HALF = M // 2
R = 512
K = HALF // R
S = 4


def kernel(partial, resid, gamma):
    my_y = lax.axis_index("y")
    mypart = lax.dynamic_slice(partial[0], (my_y * HALF, 0), (HALF, D))
    myresid = lax.dynamic_slice(resid, (my_y * HALF, 0), (HALF, D))
    gamma2 = gamma.reshape(1, D)

    def body(part_hbm, part_ref, resid_ref, gamma_ref, out_ref,
             recv_x, out_vmem, sx, rx, sy, ry, lc):
        k = pl.program_id(0)
        x = lax.axis_index("x")
        y = lax.axis_index("y")
        slot2 = lax.rem(k, 2)

        def x_rdma(j):
            return pltpu.make_async_remote_copy(
                src_ref=part_hbm.at[pl.ds(j * R, R)],
                dst_ref=recv_x.at[lax.rem(j, S)],
                send_sem=sx.at[j],
                recv_sem=rx.at[j],
                device_id=(1 - x, y),
                device_id_type=pl.DeviceIdType.MESH,
            )

        @pl.when(k == 0)
        def _():
            bsem = pltpu.get_barrier_semaphore()
            pl.semaphore_signal(bsem, inc=1, device_id=(1 - x, y),
                                device_id_type=pl.DeviceIdType.MESH)
            pl.semaphore_signal(bsem, inc=1, device_id=(x, 1 - y),
                                device_id_type=pl.DeviceIdType.MESH)
            pl.semaphore_wait(bsem, 2)
            x_rdma(0).start()
            if K > 1:
                x_rdma(1).start()

        @pl.when((k > 0) & (k < K - 1))
        def _():
            x_rdma(k + 1).start()

        x_rdma(k).wait_recv()

        @pl.when(k >= 2)
        def _():
            pltpu.make_async_remote_copy(
                src_ref=out_vmem.at[slot2],
                dst_ref=out_ref.at[pl.ds(0, R)],
                send_sem=sy.at[k - 2],
                recv_sem=ry.at[k - 2],
                device_id=(x, 1 - y),
                device_id_type=pl.DeviceIdType.MESH,
            ).wait_send()
            pltpu.make_async_copy(out_vmem.at[slot2],
                                  out_ref.at[pl.ds(0, R)], lc.at[slot2]).wait()

        yc = part_ref[...] + recv_x[lax.rem(k, S)] + resid_ref[...]
        rms = jnp.sqrt(jnp.mean(yc * yc, axis=-1, keepdims=True) + 1e-6)
        out_vmem[slot2] = yc / rms * gamma_ref[...]

        my_row = y * HALF + k * R

        pltpu.make_async_copy(out_vmem.at[slot2],
                              out_ref.at[pl.ds(my_row, R)],
                              lc.at[slot2]).start()

        pltpu.make_async_remote_copy(
            src_ref=out_vmem.at[slot2],
            dst_ref=out_ref.at[pl.ds(my_row, R)],
            send_sem=sy.at[k],
            recv_sem=ry.at[k],
            device_id=(x, 1 - y),
            device_id_type=pl.DeviceIdType.MESH,
        ).start()

        @pl.when(k == K - 1)
        def _():
            for j in range(K):
                x_rdma(j).wait_send()
            for j in (K - 2, K - 1):
                if j >= 0:
                    pltpu.make_async_remote_copy(
                        src_ref=out_vmem.at[j % 2],
                        dst_ref=out_ref.at[pl.ds(0, R)],
                        send_sem=sy.at[j],
                        recv_sem=ry.at[j],
                        device_id=(x, 1 - y),
                        device_id_type=pl.DeviceIdType.MESH,
                    ).wait_send()
                    pltpu.make_async_copy(out_vmem.at[j % 2],
                                          out_ref.at[pl.ds(0, R)],
                                          lc.at[j % 2]).wait()
            for j in range(K):
                their_row = (1 - y) * HALF + j * R
                pltpu.make_async_remote_copy(
                    src_ref=out_vmem.at[0],
                    dst_ref=out_ref.at[pl.ds(their_row, R)],
                    send_sem=sy.at[j],
                    recv_sem=ry.at[j],
                    device_id=(x, 1 - y),
                    device_id_type=pl.DeviceIdType.MESH,
                ).wait_recv()

    return pl.pallas_call(
        body,
        grid=(K,),
        in_specs=[
            pl.BlockSpec(memory_space=pl.ANY),
            pl.BlockSpec((R, D), lambda k: (k, 0)),
            pl.BlockSpec((R, D), lambda k: (k, 0)),
            pl.BlockSpec((1, D), lambda k: (0, 0)),
        ],
        out_specs=pl.BlockSpec(memory_space=pl.ANY),
        out_shape=jax.ShapeDtypeStruct((M, D), jnp.float32),
        scratch_shapes=[
            pltpu.VMEM((S, R, D), jnp.float32),
            pltpu.VMEM((2, R, D), jnp.float32),
            pltpu.SemaphoreType.DMA((K,)),
            pltpu.SemaphoreType.DMA((K,)),
            pltpu.SemaphoreType.DMA((K,)),
            pltpu.SemaphoreType.DMA((K,)),
            pltpu.SemaphoreType.DMA((2,)),
        ],
        compiler_params=pltpu.CompilerParams(
            collective_id=0,
            dimension_semantics=("arbitrary",),
            vmem_limit_bytes=64 * 1024 * 1024,
        ),
    )(mypart, mypart, myresid, gamma2)


# device time: 419384 ns/iter; 2.0347x vs baseline; 1.2081x over previous
import jax
import jax.numpy as jnp
from jax import lax
from jax.experimental import pallas as pl
from jax.experimental.pallas import tpu as pltpu

M = 8192
D = 2048
HALF = M // 2
R = 64
K = HALF // R


def kernel(partial, resid, gamma):
    gamma2 = gamma.reshape(1, D)

    def body(part_hbm, part_ref, resid_ref, gamma_ref, out_ref,
             recv_x, out_vmem, sx, rx, sy, ry, lc):
        k = pl.program_id(0)
        x = lax.axis_index("x")
        y = lax.axis_index("y")
        slot2 = lax.rem(k, 2)

        def x_rdma(j):
            return pltpu.make_async_remote_copy(
                src_ref=part_hbm.at[0, pl.ds(y * HALF + j * R, R)],
                dst_ref=recv_x.at[j],
                send_sem=sx.at[j],
                recv_sem=rx.at[j],
                device_id=(1 - x, y),
                device_id_type=pl.DeviceIdType.MESH,
            )

        @pl.when(k == 0)
        def _():
            bsem = pltpu.get_barrier_semaphore()
            pl.semaphore_signal(bsem, inc=1, device_id=(1 - x, y),
                                device_id_type=pl.DeviceIdType.MESH)
            pl.semaphore_signal(bsem, inc=1, device_id=(x, 1 - y),
                                device_id_type=pl.DeviceIdType.MESH)
            pl.semaphore_wait(bsem, 2)
            for j in range(K):
                x_rdma(j).start()

        x_rdma(k).wait_recv()

        @pl.when(k >= 2)
        def _():
            pltpu.make_async_remote_copy(
                src_ref=out_vmem.at[slot2],
                dst_ref=out_ref.at[pl.ds(0, R)],
                send_sem=sy.at[k - 2],
                recv_sem=ry.at[k - 2],
                device_id=(x, 1 - y),
                device_id_type=pl.DeviceIdType.MESH,
            ).wait_send()
            pltpu.make_async_copy(out_vmem.at[slot2],
                                  out_ref.at[pl.ds(0, R)], lc.at[slot2]).wait()

        yc = part_ref[0] + recv_x[k] + resid_ref[...]
        inv = lax.rsqrt(jnp.mean(yc * yc, axis=-1, keepdims=True) + 1e-6)
        out_vmem[slot2] = yc * inv * gamma_ref[...]

        my_row = y * HALF + k * R

        pltpu.make_async_copy(out_vmem.at[slot2],
                              out_ref.at[pl.ds(my_row, R)],
                              lc.at[slot2]).start()

        pltpu.make_async_remote_copy(
            src_ref=out_vmem.at[slot2],
            dst_ref=out_ref.at[pl.ds(my_row, R)],
            send_sem=sy.at[k],
            recv_sem=ry.at[k],
            device_id=(x, 1 - y),
            device_id_type=pl.DeviceIdType.MESH,
        ).start()

        @pl.when(k == K - 1)
        def _():
            for j in range(K):
                x_rdma(j).wait_send()
            for j in (K - 2, K - 1):
                pltpu.make_async_remote_copy(
                    src_ref=out_vmem.at[j % 2],
                    dst_ref=out_ref.at[pl.ds(0, R)],
                    send_sem=sy.at[j],
                    recv_sem=ry.at[j],
                    device_id=(x, 1 - y),
                    device_id_type=pl.DeviceIdType.MESH,
                ).wait_send()
                pltpu.make_async_copy(out_vmem.at[j % 2],
                                      out_ref.at[pl.ds(0, R)],
                                      lc.at[j % 2]).wait()
            for j in range(K):
                their_row = (1 - y) * HALF + j * R
                pltpu.make_async_remote_copy(
                    src_ref=out_vmem.at[0],
                    dst_ref=out_ref.at[pl.ds(their_row, R)],
                    send_sem=sy.at[j],
                    recv_sem=ry.at[j],
                    device_id=(x, 1 - y),
                    device_id_type=pl.DeviceIdType.MESH,
                ).wait_recv()

    return pl.pallas_call(
        body,
        grid=(K,),
        in_specs=[
            pl.BlockSpec(memory_space=pl.ANY),
            pl.BlockSpec(
                (1, R, D),
                lambda k: (0, lax.axis_index("y") * (HALF // R) + k, 0)),
            pl.BlockSpec(
                (R, D), lambda k: (lax.axis_index("y") * (HALF // R) + k, 0)),
            pl.BlockSpec((1, D), lambda k: (0, 0)),
        ],
        out_specs=pl.BlockSpec(memory_space=pl.ANY),
        out_shape=jax.ShapeDtypeStruct((M, D), jnp.float32),
        scratch_shapes=[
            pltpu.VMEM((K, R, D), jnp.float32),
            pltpu.VMEM((2, R, D), jnp.float32),
            pltpu.SemaphoreType.DMA((K,)),
            pltpu.SemaphoreType.DMA((K,)),
            pltpu.SemaphoreType.DMA((K,)),
            pltpu.SemaphoreType.DMA((K,)),
            pltpu.SemaphoreType.DMA((2,)),
        ],
        compiler_params=pltpu.CompilerParams(
            collective_id=0,
            dimension_semantics=("arbitrary",),
            vmem_limit_bytes=64 * 1024 * 1024,
        ),
    )(partial, partial, resid, gamma2)
